# baseline (speedup 1.0000x reference)
import functools
import jax
import jax.numpy as jnp
from jax import lax
from jax.experimental import pallas as pl
from jax.experimental.pallas import tpu as pltpu
from jax.experimental.pallas import tpu_sc as plsc

_Q = 32
_D = 512
_M = 100000
_MH = _M // 2
_BM = 4096
_K = 5
_SB = 1024
_NSETS = 4
_NC = _NSETS * _K * 128


def _scan(mem_ref, a, r, rbase, valid, iota):
    for g in range(_BM // _SB):
        mem_g = mem_ref[0, pl.ds(g * _SB, _SB), :]
        memb = mem_g.astype(jnp.bfloat16)
        msq = memb * memb
        bmat = jnp.concatenate([memb, msq], axis=1)
        t = lax.dot_general(a, bmat, (((1,), (1,)), ((), ())),
                            preferred_element_type=jnp.float32)
        t = jnp.where(iota < valid - g * _SB, t, jnp.inf)
        for c in range(_SB // 128):
            x = t[:, c * 128:(c + 1) * 128]
            o = rbase + (c % 2) * _K
            for k in range(_K):
                lo = jnp.minimum(r[o + k], x)
                x = jnp.maximum(r[o + k], x)
                r[o + k] = lo


def _knn_kernel(obs_ref, W_ref, b_ref, memA_ref, memB_ref, out_ref,
                a_s, emb_s, run_s):
    i = pl.program_id(0)
    nb = pl.num_programs(0)

    @pl.when(i == 0)
    def _init():
        emb = lax.dot_general(
            obs_ref[...], W_ref[...], (((1,), (0,)), ((), ())),
            preferred_element_type=jnp.float32) + b_ref[...]
        emb_s[...] = emb
        a_s[:, :_D] = (-2.0 * emb).astype(jnp.bfloat16)
        a_s[:, _D:] = jnp.ones((_Q, _D), jnp.bfloat16)
        run_s[...] = jnp.full((_Q, _NC), jnp.inf, jnp.float32)

    r = [run_s[:, k * 128:(k + 1) * 128] for k in range(_NSETS * _K)]
    valid = _MH - i * _BM
    a = a_s[...]
    iota = lax.broadcasted_iota(jnp.int32, (_Q, _SB), 1)
    _scan(memA_ref, a, r, 0, valid, iota)
    _scan(memB_ref, a, r, 2 * _K, valid, iota)
    for k in range(_NSETS * _K):
        run_s[:, k * 128:(k + 1) * 128] = r[k]

    @pl.when(i == nb - 1)
    def _fin():
        e = emb_s[...]
        q2 = jnp.sum(e * e, axis=1, keepdims=True)
        out_ref[...] = run_s[...] + q2


def _tc_scan(obs, mem3, W, b2):
    nb = pl.cdiv(_MH, _BM)
    return pl.pallas_call(
        _knn_kernel,
        grid=(nb,),
        in_specs=[
            pl.BlockSpec(obs.shape, lambda i: (0, 0)),
            pl.BlockSpec(W.shape, lambda i: (0, 0)),
            pl.BlockSpec((1, _D), lambda i: (0, 0)),
            pl.BlockSpec((1, _BM, _D), lambda i: (0, i, 0)),
            pl.BlockSpec((1, _BM, _D), lambda i: (1, i, 0)),
        ],
        out_specs=pl.BlockSpec((_Q, _NC), lambda i: (0, 0)),
        out_shape=jax.ShapeDtypeStruct((_Q, _NC), jnp.float32),
        scratch_shapes=[
            pltpu.VMEM((_Q, 2 * _D), jnp.bfloat16),
            pltpu.VMEM((_Q, _D), jnp.float32),
            pltpu.VMEM((_Q, _NC), jnp.float32),
        ],
    )(obs, W, b2, mem3, mem3)


def _sc_top5(cand):
    mesh = plsc.VectorSubcoreMesh(core_axis_name="c", subcore_axis_name="s")

    @functools.partial(
        pl.kernel, mesh=mesh,
        out_type=jax.ShapeDtypeStruct((_Q, 128), jnp.float32),
        scratch_types=[
            pltpu.VMEM((_NC,), jnp.float32),
            pltpu.VMEM((128,), jnp.float32),
        ],
    )
    def k(cand_hbm, out_hbm, row_v, res_v):
        wid = lax.axis_index("s") * 2 + lax.axis_index("c")
        pltpu.sync_copy(cand_hbm.at[wid], row_v)

        inf16 = jnp.full((16,), jnp.inf, jnp.float32)

        def body(c, rs):
            x = row_v[pl.ds(c * 16, 16)]
            out = []
            for k5 in range(_K):
                lo = jnp.minimum(rs[k5], x)
                x = jnp.maximum(rs[k5], x)
                out.append(lo)
            return tuple(out)

        rs = lax.fori_loop(0, _NC // 16, body,
                           (inf16, inf16, inf16, inf16, inf16))
        for k5 in range(_K):
            res_v[pl.ds(k5 * 16, 16)] = rs[k5]
        for k5 in range(_K, 8):
            res_v[pl.ds(k5 * 16, 16)] = inf16
        pltpu.sync_copy(res_v, out_hbm.at[wid])

    return k(cand)


def _final_kernel(c_ref, out_ref):
    cand = c_ref[...]
    acc = jnp.zeros((_Q, 1), jnp.float32)
    for _ in range(_K):
        m = jnp.min(cand, axis=1, keepdims=True)
        cand = jnp.where(cand == m, jnp.inf, cand)
        acc = acc + jnp.sqrt(jnp.maximum(m, 0.0) + 1e-12)
    out_ref[0, 0] = jnp.sum(acc) / (_Q * _K)


def _tc_final(c):
    return pl.pallas_call(
        _final_kernel,
        out_specs=pl.BlockSpec((1, 1), memory_space=pltpu.SMEM),
        out_shape=jax.ShapeDtypeStruct((1, 1), jnp.float32),
    )(c)


def kernel(obs, memory, W, b):
    b2 = b.reshape(1, _D)
    mem3 = memory.reshape(2, _MH, _D)
    cand = _tc_scan(obs, mem3, W, b2)
    top5 = _sc_top5(cand)
    return _tc_final(top5)[0, 0]


# speedup vs baseline: 1.2610x; 1.2610x over previous
import jax
import jax.numpy as jnp
from jax import lax
from jax.experimental import pallas as pl
from jax.experimental.pallas import tpu as pltpu

_Q = 32
_D = 512
_M = 100000
_BM = 8192
_K = 5
_SB = 1024
_NSETS = 2


def _knn_kernel(obs_ref, W_ref, b_ref, mem_ref, out_ref, a_s, emb_s, run_s):
    i = pl.program_id(0)
    nb = pl.num_programs(0)

    @pl.when(i == 0)
    def _init():
        emb = lax.dot_general(
            obs_ref[...], W_ref[...], (((1,), (0,)), ((), ())),
            preferred_element_type=jnp.float32) + b_ref[...]
        emb_s[...] = emb
        a_s[:, :_D] = (-2.0 * emb).astype(jnp.bfloat16)
        a_s[:, _D:] = jnp.ones((_Q, _D), jnp.bfloat16)
        run_s[...] = jnp.full((_Q, _NSETS * _K * 128), jnp.inf, jnp.float32)

    r = [run_s[:, k * 128:(k + 1) * 128] for k in range(_NSETS * _K)]
    valid = _M - i * _BM
    a = a_s[...]
    iota = lax.broadcasted_iota(jnp.int32, (_Q, _SB), 1)
    for g in range(_BM // _SB):
        mem_g = mem_ref[pl.ds(g * _SB, _SB), :]
        memb = mem_g.astype(jnp.bfloat16)
        msq = memb * memb
        bmat = jnp.concatenate([memb, msq], axis=1)
        t = lax.dot_general(a, bmat, (((1,), (1,)), ((), ())),
                            preferred_element_type=jnp.float32)
        t = jnp.where(iota < valid - g * _SB, t, jnp.inf)
        for c in range(_SB // 128):
            x = t[:, c * 128:(c + 1) * 128]
            o = (c % 2) * _K
            for k in range(_K):
                lo = jnp.minimum(r[o + k], x)
                x = jnp.maximum(r[o + k], x)
                r[o + k] = lo
    for k in range(_NSETS * _K):
        run_s[:, k * 128:(k + 1) * 128] = r[k]

    @pl.when(i == nb - 1)
    def _fin():
        e = emb_s[...]
        q2 = jnp.sum(e * e, axis=1, keepdims=True)
        cand = run_s[...]
        acc = jnp.zeros((_Q, 1), jnp.float32)
        for _ in range(_K):
            m = jnp.min(cand, axis=1, keepdims=True)
            cand = jnp.where(cand == m, jnp.inf, cand)
            acc = acc + jnp.sqrt(jnp.maximum(m + q2, 0.0) + 1e-12)
        out_ref[0, 0] = jnp.sum(acc) / (_Q * _K)


def kernel(obs, memory, W, b):
    nb = pl.cdiv(_M, _BM)
    b2 = b.reshape(1, _D)
    out = pl.pallas_call(
        _knn_kernel,
        grid=(nb,),
        in_specs=[
            pl.BlockSpec(obs.shape, lambda i: (0, 0)),
            pl.BlockSpec(W.shape, lambda i: (0, 0)),
            pl.BlockSpec((1, _D), lambda i: (0, 0)),
            pl.BlockSpec((_BM, _D), lambda i: (i, 0)),
        ],
        out_specs=pl.BlockSpec((1, 1), lambda i: (0, 0),
                               memory_space=pltpu.SMEM),
        out_shape=jax.ShapeDtypeStruct((1, 1), jnp.float32),
        scratch_shapes=[
            pltpu.VMEM((_Q, 2 * _D), jnp.bfloat16),
            pltpu.VMEM((_Q, _D), jnp.float32),
            pltpu.VMEM((_Q, _NSETS * _K * 128), jnp.float32),
        ],
    )(obs, W, b2, memory)
    return out[0, 0]


# speedup vs baseline: 1.3006x; 1.0314x over previous
import jax
import jax.numpy as jnp
from jax import lax
from jax.experimental import pallas as pl
from jax.experimental.pallas import tpu as pltpu

_Q = 32
_D = 512
_M = 100000
_BM = 10240
_K = 5
_SB = 1024
_NSETS = 2


def _knn_kernel(obs_ref, W_ref, b_ref, mem_ref, out_ref, a_s, emb_s, run_s):
    i = pl.program_id(0)
    nb = pl.num_programs(0)

    @pl.when(i == 0)
    def _init():
        emb = lax.dot_general(
            obs_ref[...], W_ref[...], (((1,), (0,)), ((), ())),
            preferred_element_type=jnp.float32) + b_ref[...]
        emb_s[...] = emb
        a_s[:, :_D] = (-2.0 * emb).astype(jnp.bfloat16)
        a_s[:, _D:] = jnp.ones((_Q, _D), jnp.bfloat16)
        run_s[...] = jnp.full((_Q, _NSETS * _K * 128), jnp.inf, jnp.float32)

    r = [run_s[:, k * 128:(k + 1) * 128] for k in range(_NSETS * _K)]
    valid = _M - i * _BM
    a = a_s[...]
    iota = lax.broadcasted_iota(jnp.int32, (_Q, _SB), 1)
    for g in range(_BM // _SB):
        mem_g = mem_ref[pl.ds(g * _SB, _SB), :]
        memb = mem_g.astype(jnp.bfloat16)
        msq = memb * memb
        bmat = jnp.concatenate([memb, msq], axis=1)
        t = lax.dot_general(a, bmat, (((1,), (1,)), ((), ())),
                            preferred_element_type=jnp.float32)
        t = jnp.where(iota < valid - g * _SB, t, jnp.inf)
        for c in range(_SB // 128):
            x = t[:, c * 128:(c + 1) * 128]
            o = (c % 2) * _K
            for k in range(_K):
                lo = jnp.minimum(r[o + k], x)
                x = jnp.maximum(r[o + k], x)
                r[o + k] = lo
    for k in range(_NSETS * _K):
        run_s[:, k * 128:(k + 1) * 128] = r[k]

    @pl.when(i == nb - 1)
    def _fin():
        e = emb_s[...]
        q2 = jnp.sum(e * e, axis=1, keepdims=True)
        cand = run_s[...]
        acc = jnp.zeros((_Q, 1), jnp.float32)
        for _ in range(_K):
            m = jnp.min(cand, axis=1, keepdims=True)
            cand = jnp.where(cand == m, jnp.inf, cand)
            acc = acc + jnp.sqrt(jnp.maximum(m + q2, 0.0) + 1e-12)
        out_ref[0, 0] = jnp.sum(acc) / (_Q * _K)


def kernel(obs, memory, W, b):
    nb = pl.cdiv(_M, _BM)
    b2 = b.reshape(1, _D)
    out = pl.pallas_call(
        _knn_kernel,
        grid=(nb,),
        in_specs=[
            pl.BlockSpec(obs.shape, lambda i: (0, 0)),
            pl.BlockSpec(W.shape, lambda i: (0, 0)),
            pl.BlockSpec((1, _D), lambda i: (0, 0)),
            pl.BlockSpec((_BM, _D), lambda i: (i, 0)),
        ],
        out_specs=pl.BlockSpec((1, 1), lambda i: (0, 0),
                               memory_space=pltpu.SMEM),
        out_shape=jax.ShapeDtypeStruct((1, 1), jnp.float32),
        scratch_shapes=[
            pltpu.VMEM((_Q, 2 * _D), jnp.bfloat16),
            pltpu.VMEM((_Q, _D), jnp.float32),
            pltpu.VMEM((_Q, _NSETS * _K * 128), jnp.float32),
        ],
    )(obs, W, b2, memory)
    return out[0, 0]


# speedup vs baseline: 1.3190x; 1.0142x over previous
import jax
import jax.numpy as jnp
from jax import lax
from jax.experimental import pallas as pl
from jax.experimental.pallas import tpu as pltpu

_Q = 32
_D = 512
_M = 100000
_BM = 12800
_K = 5
_SB = 1024
_NSETS = 2


def _knn_kernel(obs_ref, W_ref, b_ref, mem_ref, out_ref, a_s, emb_s, run_s):
    i = pl.program_id(0)
    nb = pl.num_programs(0)

    @pl.when(i == 0)
    def _init():
        emb = lax.dot_general(
            obs_ref[...], W_ref[...], (((1,), (0,)), ((), ())),
            preferred_element_type=jnp.float32) + b_ref[...]
        emb_s[...] = emb
        a_s[:, :_D] = (-2.0 * emb).astype(jnp.bfloat16)
        a_s[:, _D:] = jnp.ones((_Q, _D), jnp.bfloat16)
        run_s[...] = jnp.full((_Q, _NSETS * _K * 128), jnp.inf, jnp.float32)

    r = [run_s[:, k * 128:(k + 1) * 128] for k in range(_NSETS * _K)]
    valid = _M - i * _BM
    a = a_s[...]
    iota = lax.broadcasted_iota(jnp.int32, (_Q, _SB), 1)
    for g in range(_BM // _SB):
        mem_g = mem_ref[pl.ds(g * _SB, _SB), :]
        memb = mem_g.astype(jnp.bfloat16)
        msq = memb * memb
        bmat = jnp.concatenate([memb, msq], axis=1)
        t = lax.dot_general(a, bmat, (((1,), (1,)), ((), ())),
                            preferred_element_type=jnp.float32)
        t = jnp.where(iota < valid - g * _SB, t, jnp.inf)
        for c in range(_SB // 128):
            x = t[:, c * 128:(c + 1) * 128]
            o = (c % 2) * _K
            for k in range(_K):
                lo = jnp.minimum(r[o + k], x)
                x = jnp.maximum(r[o + k], x)
                r[o + k] = lo
    for k in range(_NSETS * _K):
        run_s[:, k * 128:(k + 1) * 128] = r[k]

    @pl.when(i == nb - 1)
    def _fin():
        e = emb_s[...]
        q2 = jnp.sum(e * e, axis=1, keepdims=True)
        cand = run_s[...]
        acc = jnp.zeros((_Q, 1), jnp.float32)
        for _ in range(_K):
            m = jnp.min(cand, axis=1, keepdims=True)
            cand = jnp.where(cand == m, jnp.inf, cand)
            acc = acc + jnp.sqrt(jnp.maximum(m + q2, 0.0) + 1e-12)
        out_ref[0, 0] = jnp.sum(acc) / (_Q * _K)


def kernel(obs, memory, W, b):
    nb = pl.cdiv(_M, _BM)
    b2 = b.reshape(1, _D)
    out = pl.pallas_call(
        _knn_kernel,
        grid=(nb,),
        in_specs=[
            pl.BlockSpec(obs.shape, lambda i: (0, 0)),
            pl.BlockSpec(W.shape, lambda i: (0, 0)),
            pl.BlockSpec((1, _D), lambda i: (0, 0)),
            pl.BlockSpec((_BM, _D), lambda i: (i, 0)),
        ],
        out_specs=pl.BlockSpec((1, 1), lambda i: (0, 0),
                               memory_space=pltpu.SMEM),
        out_shape=jax.ShapeDtypeStruct((1, 1), jnp.float32),
        scratch_shapes=[
            pltpu.VMEM((_Q, 2 * _D), jnp.bfloat16),
            pltpu.VMEM((_Q, _D), jnp.float32),
            pltpu.VMEM((_Q, _NSETS * _K * 128), jnp.float32),
        ],
    )(obs, W, b2, memory)
    return out[0, 0]
